# baseline (speedup 1.0000x reference)
import functools

import jax
import jax.numpy as jnp
from jax import lax
from jax.experimental import pallas as pl
from jax.experimental.pallas import tpu as pltpu
from jax.experimental.pallas import tpu_sc as plsc

NV = 10000
NCL = 10000
EP = 160000
D = 128
NPAD = 10240
BR = 512
GRID = NPAD // BR
NCORES = 2
NSUB = 16
EPPAD = 163840
CHUNK = 128
NCHUNK_L = EPPAD // CHUNK
SEC = 16
NSEC_PAIR = NCHUNK_L // (NSUB * SEC)
ABASE = 4
KSPLIT = 16
ACC_ROWS = 10112
TPT = ACC_ROWS // NSUB
CNT_R = 80
PAD_IDX = 10000
NBUF = 2


def _mm_body(x_ref, wp_ref, bp_ref, wn_ref, bn_ref, tp_ref, tn_ref):
    x = x_ref[...]
    tp_ref[...] = (
        jnp.dot(x, wp_ref[...], preferred_element_type=jnp.float32) + bp_ref[...]
    )
    tn_ref[...] = (
        jnp.dot(x, wn_ref[...], preferred_element_type=jnp.float32) + bn_ref[...]
    )


def _tables_from_x(x_pad, wp, bp, wn, bn):
    return pl.pallas_call(
        _mm_body,
        grid=(GRID,),
        in_specs=[
            pl.BlockSpec((BR, D), lambda i: (i, 0)),
            pl.BlockSpec((D, D), lambda i: (0, 0)),
            pl.BlockSpec((1, D), lambda i: (0, 0)),
            pl.BlockSpec((D, D), lambda i: (0, 0)),
            pl.BlockSpec((1, D), lambda i: (0, 0)),
        ],
        out_specs=[
            pl.BlockSpec((BR, D), lambda i: (i, 0)),
            pl.BlockSpec((BR, D), lambda i: (i, 0)),
        ],
        out_shape=[
            jax.ShapeDtypeStruct((NPAD, D), jnp.float32),
            jax.ShapeDtypeStruct((NPAD, D), jnp.float32),
        ],
    )(x_pad, wp, bp, wn, bn)


def _deg_col(deg_ref):
    dblk = BR // CHUNK
    deg = jnp.sum(deg_ref[...], axis=0).reshape(dblk, CHUNK)
    deg = jnp.where(deg == 0.0, 1.0, deg)
    rid = lax.broadcasted_iota(jnp.int32, (BR, 1), 0)
    sel = (lax.broadcasted_iota(jnp.int32, (BR, dblk), 1)
           == (rid >> 7)).astype(jnp.float32)
    tmp = jnp.dot(sel, deg, preferred_element_type=jnp.float32)
    msk = (lax.broadcasted_iota(jnp.int32, (BR, CHUNK), 1)
           == (rid & 127)).astype(jnp.float32)
    return jnp.sum(tmp * msk, axis=1, keepdims=True)


def _combine_mm_body(ap_ref, an_ref, deg_ref, xc_ref, wp_ref, rp_ref, bp_ref,
                     wn_ref, rn_ref, bn_ref, tp_ref, tn_ref):
    ssum = ap_ref[...] + an_ref[...]
    deg = _deg_col(deg_ref)
    cemb = jnp.maximum(ssum / deg, 0.0)
    xc = xc_ref[...][:, :1]
    tp_ref[...] = (
        jnp.dot(cemb, wp_ref[...], preferred_element_type=jnp.float32)
        + xc * rp_ref[...] + bp_ref[...]
    )
    tn_ref[...] = (
        jnp.dot(cemb, wn_ref[...], preferred_element_type=jnp.float32)
        + xc * rn_ref[...] + bn_ref[...]
    )


def _tables_from_acc(acc2, deg, xc_pad, wp, rp, bp, wn, rn, bn):
    dblk = BR // CHUNK
    return pl.pallas_call(
        _combine_mm_body,
        grid=(GRID,),
        in_specs=[
            pl.BlockSpec((BR, D), lambda i: (i, 0)),
            pl.BlockSpec((BR, D), lambda i: (i, 0)),
            pl.BlockSpec((2 * NSUB, 1, dblk, CHUNK), lambda i: (0, i, 0, 0)),
            pl.BlockSpec((BR, 8), lambda i: (i, 0)),
            pl.BlockSpec((D, D), lambda i: (0, 0)),
            pl.BlockSpec((1, D), lambda i: (0, 0)),
            pl.BlockSpec((1, D), lambda i: (0, 0)),
            pl.BlockSpec((D, D), lambda i: (0, 0)),
            pl.BlockSpec((1, D), lambda i: (0, 0)),
            pl.BlockSpec((1, D), lambda i: (0, 0)),
        ],
        out_specs=[
            pl.BlockSpec((BR, D), lambda i: (i, 0)),
            pl.BlockSpec((BR, D), lambda i: (i, 0)),
        ],
        out_shape=[
            jax.ShapeDtypeStruct((NPAD, D), jnp.float32),
            jax.ShapeDtypeStruct((NPAD, D), jnp.float32),
        ],
    )(acc2[0], acc2[1], deg, xc_pad, wp, rp, bp, wn, rn, bn)


def _final_body(ap_ref, an_ref, deg_ref, o_ref):
    ssum = ap_ref[...] + an_ref[...]
    deg = _deg_col(deg_ref)
    o_ref[...] = jnp.maximum(ssum / deg, 0.0)


def _final(acc2, deg):
    dblk = BR // CHUNK
    return pl.pallas_call(
        _final_body,
        grid=(GRID,),
        in_specs=[
            pl.BlockSpec((BR, D), lambda i: (i, 0)),
            pl.BlockSpec((BR, D), lambda i: (i, 0)),
            pl.BlockSpec((2 * NSUB, 1, dblk, CHUNK), lambda i: (0, i, 0, 0)),
        ],
        out_specs=pl.BlockSpec((BR, D), lambda i: (i, 0)),
        out_shape=jax.ShapeDtypeStruct((NPAD, D), jnp.float32),
    )(acc2[0], acc2[1], deg)


def _seg_body(tp_hbm, tn_hbm, gp_hbm, sp_hbm, gn_hbm, sn_hbm, z_hbm,
              out_hbm, deg_hbm, gidx_v, sidx_v, cnt_v, *bufs):
    rows = bufs[:NBUF]
    acc_sh = bufs[NBUF]
    sg = bufs[NBUF + 1:NBUF + 1 + NBUF]
    ss = bufs[NBUF + 1 + NBUF:NBUF + 1 + 2 * NBUF]
    c = lax.axis_index("c")
    s = lax.axis_index("s")

    ones16 = jnp.full((16,), 1.0, dtype=jnp.float32)

    def count_chunk(j):
        for k in range(CHUNK // 16):
            idx16 = sidx_v[j, pl.ds(k * 16, 16)]
            row = lax.shift_right_logical(idx16, 7)
            col = lax.bitwise_and(idx16, 127)
            plsc.addupdate_scatter(cnt_v, [row, col], ones16)

    pltpu.sync_copy(z_hbm, rows[0])
    pltpu.sync_copy(z_hbm.at[pl.ds(0, CNT_R)], cnt_v)
    base0 = s * TPT
    tail = TPT % CHUNK

    @pl.loop(0, TPT // CHUNK)
    def _zero(r):
        pltpu.sync_copy(
            rows[0], acc_sh.at[pl.ds(base0 + r * CHUNK, CHUNK)])

    pltpu.sync_copy(
        rows[0].at[pl.ds(0, tail)],
        acc_sh.at[pl.ds(base0 + (TPT // CHUNK) * CHUNK, tail)])

    plsc.subcore_barrier()

    smk = jnp.minimum(s, KSPLIT)
    n0 = jnp.where(s < KSPLIT, ABASE + 1, ABASE)
    sec0 = ABASE * s + smk
    sec1 = (ABASE * NSUB + KSPLIT) + ((NSEC_PAIR - ABASE) * s - smk)
    nsec = jnp.where(c == 0, n0, NSEC_PAIR - n0)
    cbase = SEC * jnp.where(c == 0, sec0, sec1)

    def run_list(table_hbm, g_hbm, s_hbm):
        @pl.loop(0, nsec)
        def _sec(t):
            cb = cbase + t * SEC
            pltpu.sync_copy(g_hbm.at[pl.ds(cb, SEC)], gidx_v)
            pltpu.sync_copy(s_hbm.at[pl.ds(cb, SEC)], sidx_v)
            for b in range(NBUF):
                pltpu.async_copy(table_hbm.at[gidx_v.at[b]], rows[b], sg[b])

            @pl.loop(0, SEC - NBUF, step=NBUF)
            def _go(j):
                for b in range(NBUF):
                    pltpu.make_async_copy(
                        table_hbm.at[gidx_v.at[j + b]], rows[b], sg[b]).wait()
                    pltpu.async_copy(
                        rows[b], acc_sh.at[sidx_v.at[j + b]], ss[b], add=True)
                    count_chunk(j + b)
                for b in range(NBUF):
                    pltpu.make_async_copy(
                        rows[b], acc_sh.at[sidx_v.at[j + b]], ss[b]).wait()
                    pltpu.async_copy(
                        table_hbm.at[gidx_v.at[j + NBUF + b]], rows[b], sg[b])

            j0 = SEC - NBUF
            for b in range(NBUF):
                pltpu.make_async_copy(
                    table_hbm.at[gidx_v.at[j0 + b]], rows[b], sg[b]).wait()
                pltpu.async_copy(
                    rows[b], acc_sh.at[sidx_v.at[j0 + b]], ss[b], add=True)
                count_chunk(j0 + b)
            for b in range(NBUF):
                pltpu.make_async_copy(
                    rows[b], acc_sh.at[sidx_v.at[j0 + b]], ss[b]).wait()

    run_list(tp_hbm, gp_hbm, sp_hbm)
    run_list(tn_hbm, gn_hbm, sn_hbm)

    pltpu.sync_copy(cnt_v, deg_hbm.at[c].at[s])

    plsc.subcore_barrier()

    @pl.loop(0, TPT // CHUNK)
    def _wout(r):
        base = base0 + r * CHUNK
        pltpu.sync_copy(acc_sh.at[pl.ds(base, CHUNK)], rows[0])
        pltpu.sync_copy(rows[0], out_hbm.at[c].at[pl.ds(base, CHUNK)])

    tb = base0 + (TPT // CHUNK) * CHUNK
    pltpu.sync_copy(acc_sh.at[pl.ds(tb, tail)], rows[0].at[pl.ds(0, tail)])
    pltpu.sync_copy(
        rows[0].at[pl.ds(0, tail)], out_hbm.at[c].at[pl.ds(tb, tail)])


@functools.cache
def _make_seg_pass():
    return pl.kernel(
        _seg_body,
        out_type=[
            jax.ShapeDtypeStruct((NCORES, NPAD, D), jnp.float32),
            jax.ShapeDtypeStruct((NCORES, NSUB, CNT_R, CHUNK), jnp.float32),
        ],
        mesh=plsc.VectorSubcoreMesh(
            core_axis_name="c", subcore_axis_name="s",
            num_cores=NCORES, num_subcores=NSUB),
        scratch_types=[
            pltpu.VMEM((SEC, CHUNK), jnp.int32),
            pltpu.VMEM((SEC, CHUNK), jnp.int32),
            pltpu.VMEM((CNT_R, CHUNK), jnp.float32),
            *([pltpu.VMEM((CHUNK, D), jnp.float32)] * NBUF),
            pltpu.VMEM_SHARED((ACC_ROWS, D), jnp.float32),
            *([pltpu.SemaphoreType.DMA] * (2 * NBUF)),
        ],
        compiler_params=pltpu.CompilerParams(needs_layout_passes=False),
    )


def _seg_pass(tp, tn, gp, sp, gn, sn, zeros):
    return _make_seg_pass()(tp, tn, gp, sp, gn, sn, zeros)


def _pad_idx(idx):
    full = jnp.concatenate(
        [idx, jnp.full((EPPAD - EP,), PAD_IDX, dtype=jnp.int32)])
    return full.reshape(NCHUNK_L, CHUNK)


def kernel(x_var, x_clause, ei_pos, ei_neg, W0vp, b0vp, W0vn, b0vn, W0cp,
           b0cp, W0cn, b0cn, W1vp, b1vp, W1vn, b1vn, W1cp, b1cp, W1cn, b1cn):
    srcP = _pad_idx(ei_pos[0])
    dstP = _pad_idx(ei_pos[1])
    srcN = _pad_idx(ei_neg[0])
    dstN = _pad_idx(ei_neg[1])
    zeros = jnp.zeros((CHUNK, D), jnp.float32)

    x_pad = jnp.pad(x_var, ((0, NPAD - NV), (0, 0)))
    xc_pad = jnp.pad(x_clause, ((0, NPAD - NCL), (0, 8 - x_clause.shape[1])))

    t_p, t_n = _tables_from_x(
        x_pad, W1vp, b1vp.reshape(1, D), W1vn, b1vn.reshape(1, D))
    acc_c, deg_c = _seg_pass(t_p, t_n, srcP, dstP, srcN, dstN, zeros)

    t2_p, t2_n = _tables_from_acc(
        acc_c, deg_c.reshape(2 * NSUB, GRID, BR // CHUNK, CHUNK), xc_pad,
        W1cp[:D], W1cp[D:D + 1], b1cp.reshape(1, D),
        W1cn[:D], W1cn[D:D + 1], b1cn.reshape(1, D))
    acc_v, deg_v = _seg_pass(t2_p, t2_n, dstP, srcP, dstN, srcN, zeros)

    return _final(
        acc_v, deg_v.reshape(2 * NSUB, GRID, BR // CHUNK, CHUNK))[:NV]


# speedup vs baseline: 1.3060x; 1.3060x over previous
import functools

import jax
import jax.numpy as jnp
from jax import lax
from jax.experimental import pallas as pl
from jax.experimental.pallas import tpu as pltpu
from jax.experimental.pallas import tpu_sc as plsc

NV = 10000
NCL = 10000
EP = 160000
D = 128
NPAD = 10240
BR = 512
GRID = NPAD // BR
NCORES = 2
NSUB = 16
EPPAD = 163840
CHUNK = 128
NCHUNK_L = EPPAD // CHUNK
SEC = 16
NSEC_PAIR = NCHUNK_L // (NSUB * SEC)
ABASE = 4
KSPLIT = 14
ACC_ROWS = 10112
TPT = ACC_ROWS // NSUB
CNT_R = 80
PAD_IDX = 10000
NBUF = 2


def _mm_body(x_ref, wp_ref, bp_ref, wn_ref, bn_ref, tp_ref, tn_ref):
    x = x_ref[...]
    tp_ref[...] = (
        jnp.dot(x, wp_ref[...], preferred_element_type=jnp.float32) + bp_ref[...]
    )
    tn_ref[...] = (
        jnp.dot(x, wn_ref[...], preferred_element_type=jnp.float32) + bn_ref[...]
    )


def _tables_from_x(x_pad, wp, bp, wn, bn):
    return pl.pallas_call(
        _mm_body,
        grid=(GRID,),
        in_specs=[
            pl.BlockSpec((BR, D), lambda i: (i, 0)),
            pl.BlockSpec((D, D), lambda i: (0, 0)),
            pl.BlockSpec((1, D), lambda i: (0, 0)),
            pl.BlockSpec((D, D), lambda i: (0, 0)),
            pl.BlockSpec((1, D), lambda i: (0, 0)),
        ],
        out_specs=[
            pl.BlockSpec((BR, D), lambda i: (i, 0)),
            pl.BlockSpec((BR, D), lambda i: (i, 0)),
        ],
        out_shape=[
            jax.ShapeDtypeStruct((NPAD, D), jnp.float32),
            jax.ShapeDtypeStruct((NPAD, D), jnp.float32),
        ],
    )(x_pad, wp, bp, wn, bn)


def _deg_col(deg_ref):
    dblk = BR // CHUNK
    deg = jnp.sum(deg_ref[...], axis=0).reshape(dblk, CHUNK)
    deg = jnp.where(deg == 0.0, 1.0, deg)
    rid = lax.broadcasted_iota(jnp.int32, (BR, 1), 0)
    sel = (lax.broadcasted_iota(jnp.int32, (BR, dblk), 1)
           == (rid >> 7)).astype(jnp.float32)
    tmp = jnp.dot(sel, deg, preferred_element_type=jnp.float32)
    msk = (lax.broadcasted_iota(jnp.int32, (BR, CHUNK), 1)
           == (rid & 127)).astype(jnp.float32)
    return jnp.sum(tmp * msk, axis=1, keepdims=True)


def _combine_mm_body(ap_ref, an_ref, deg_ref, xc_ref, wp_ref, rp_ref, bp_ref,
                     wn_ref, rn_ref, bn_ref, tp_ref, tn_ref):
    ssum = ap_ref[...] + an_ref[...]
    deg = _deg_col(deg_ref)
    cemb = jnp.maximum(ssum / deg, 0.0)
    xc = xc_ref[...][:, :1]
    tp_ref[...] = (
        jnp.dot(cemb, wp_ref[...], preferred_element_type=jnp.float32)
        + xc * rp_ref[...] + bp_ref[...]
    )
    tn_ref[...] = (
        jnp.dot(cemb, wn_ref[...], preferred_element_type=jnp.float32)
        + xc * rn_ref[...] + bn_ref[...]
    )


def _tables_from_acc(acc2, deg, xc_pad, wp, rp, bp, wn, rn, bn):
    dblk = BR // CHUNK
    return pl.pallas_call(
        _combine_mm_body,
        grid=(GRID,),
        in_specs=[
            pl.BlockSpec((BR, D), lambda i: (i, 0)),
            pl.BlockSpec((BR, D), lambda i: (i, 0)),
            pl.BlockSpec((2 * NSUB, 1, dblk, CHUNK), lambda i: (0, i, 0, 0)),
            pl.BlockSpec((BR, 8), lambda i: (i, 0)),
            pl.BlockSpec((D, D), lambda i: (0, 0)),
            pl.BlockSpec((1, D), lambda i: (0, 0)),
            pl.BlockSpec((1, D), lambda i: (0, 0)),
            pl.BlockSpec((D, D), lambda i: (0, 0)),
            pl.BlockSpec((1, D), lambda i: (0, 0)),
            pl.BlockSpec((1, D), lambda i: (0, 0)),
        ],
        out_specs=[
            pl.BlockSpec((BR, D), lambda i: (i, 0)),
            pl.BlockSpec((BR, D), lambda i: (i, 0)),
        ],
        out_shape=[
            jax.ShapeDtypeStruct((NPAD, D), jnp.float32),
            jax.ShapeDtypeStruct((NPAD, D), jnp.float32),
        ],
    )(acc2[0], acc2[1], deg, xc_pad, wp, rp, bp, wn, rn, bn)


def _final_body(ap_ref, an_ref, deg_ref, o_ref):
    ssum = ap_ref[...] + an_ref[...]
    deg = _deg_col(deg_ref)
    o_ref[...] = jnp.maximum(ssum / deg, 0.0)


def _final(acc2, deg):
    dblk = BR // CHUNK
    return pl.pallas_call(
        _final_body,
        grid=(GRID,),
        in_specs=[
            pl.BlockSpec((BR, D), lambda i: (i, 0)),
            pl.BlockSpec((BR, D), lambda i: (i, 0)),
            pl.BlockSpec((2 * NSUB, 1, dblk, CHUNK), lambda i: (0, i, 0, 0)),
        ],
        out_specs=pl.BlockSpec((BR, D), lambda i: (i, 0)),
        out_shape=jax.ShapeDtypeStruct((NPAD, D), jnp.float32),
    )(acc2[0], acc2[1], deg)


def _seg_body(tp_hbm, tn_hbm, gp_hbm, sp_hbm, gn_hbm, sn_hbm, z_hbm,
              out_hbm, deg_hbm, gidx_v, sidx_v, cnt_v, *bufs):
    rows = bufs[:NBUF]
    acc_sh = bufs[NBUF]
    sg = bufs[NBUF + 1:NBUF + 1 + NBUF]
    ss = bufs[NBUF + 1 + NBUF:NBUF + 1 + 2 * NBUF]
    c = lax.axis_index("c")
    s = lax.axis_index("s")

    ones16 = jnp.full((16,), 1.0, dtype=jnp.float32)

    def count_chunk(j):
        for k in range(CHUNK // 16):
            idx16 = sidx_v[j, pl.ds(k * 16, 16)]
            row = lax.shift_right_logical(idx16, 7)
            col = lax.bitwise_and(idx16, 127)
            plsc.addupdate_scatter(cnt_v, [row, col], ones16)

    pltpu.sync_copy(z_hbm, rows[0])
    pltpu.sync_copy(z_hbm.at[pl.ds(0, CNT_R)], cnt_v)
    base0 = s * TPT
    tail = TPT % CHUNK

    @pl.loop(0, TPT // CHUNK)
    def _zero(r):
        pltpu.sync_copy(
            rows[0], acc_sh.at[pl.ds(base0 + r * CHUNK, CHUNK)])

    pltpu.sync_copy(
        rows[0].at[pl.ds(0, tail)],
        acc_sh.at[pl.ds(base0 + (TPT // CHUNK) * CHUNK, tail)])

    plsc.subcore_barrier()

    smk = jnp.minimum(s, KSPLIT)
    n0 = jnp.where(s < KSPLIT, ABASE + 1, ABASE)
    sec0 = ABASE * s + smk
    sec1 = (ABASE * NSUB + KSPLIT) + ((NSEC_PAIR - ABASE) * s - smk)
    nsec = jnp.where(c == 0, n0, NSEC_PAIR - n0)
    cbase = SEC * jnp.where(c == 0, sec0, sec1)

    def run_list(table_hbm, g_hbm, s_hbm):
        @pl.loop(0, nsec)
        def _sec(t):
            cb = cbase + t * SEC
            pltpu.sync_copy(g_hbm.at[pl.ds(cb, SEC)], gidx_v)
            pltpu.sync_copy(s_hbm.at[pl.ds(cb, SEC)], sidx_v)
            for b in range(NBUF):
                pltpu.async_copy(table_hbm.at[gidx_v.at[b]], rows[b], sg[b])

            @pl.loop(0, SEC - NBUF, step=NBUF)
            def _go(j):
                for b in range(NBUF):
                    pltpu.make_async_copy(
                        table_hbm.at[gidx_v.at[j + b]], rows[b], sg[b]).wait()
                    pltpu.async_copy(
                        rows[b], acc_sh.at[sidx_v.at[j + b]], ss[b], add=True)
                    count_chunk(j + b)
                for b in range(NBUF):
                    pltpu.make_async_copy(
                        rows[b], acc_sh.at[sidx_v.at[j + b]], ss[b]).wait()
                    pltpu.async_copy(
                        table_hbm.at[gidx_v.at[j + NBUF + b]], rows[b], sg[b])

            j0 = SEC - NBUF
            for b in range(NBUF):
                pltpu.make_async_copy(
                    table_hbm.at[gidx_v.at[j0 + b]], rows[b], sg[b]).wait()
                pltpu.async_copy(
                    rows[b], acc_sh.at[sidx_v.at[j0 + b]], ss[b], add=True)
                count_chunk(j0 + b)
            for b in range(NBUF):
                pltpu.make_async_copy(
                    rows[b], acc_sh.at[sidx_v.at[j0 + b]], ss[b]).wait()

    run_list(tp_hbm, gp_hbm, sp_hbm)
    run_list(tn_hbm, gn_hbm, sn_hbm)

    pltpu.sync_copy(cnt_v, deg_hbm.at[c].at[s])

    plsc.subcore_barrier()

    @pl.loop(0, TPT // CHUNK)
    def _wout(r):
        base = base0 + r * CHUNK
        pltpu.sync_copy(acc_sh.at[pl.ds(base, CHUNK)], rows[0])
        pltpu.sync_copy(rows[0], out_hbm.at[c].at[pl.ds(base, CHUNK)])

    tb = base0 + (TPT // CHUNK) * CHUNK
    pltpu.sync_copy(acc_sh.at[pl.ds(tb, tail)], rows[0].at[pl.ds(0, tail)])
    pltpu.sync_copy(
        rows[0].at[pl.ds(0, tail)], out_hbm.at[c].at[pl.ds(tb, tail)])


@functools.cache
def _make_seg_pass():
    return pl.kernel(
        _seg_body,
        out_type=[
            jax.ShapeDtypeStruct((NCORES, NPAD, D), jnp.float32),
            jax.ShapeDtypeStruct((NCORES, NSUB, CNT_R, CHUNK), jnp.float32),
        ],
        mesh=plsc.VectorSubcoreMesh(
            core_axis_name="c", subcore_axis_name="s",
            num_cores=NCORES, num_subcores=NSUB),
        scratch_types=[
            pltpu.VMEM((SEC, CHUNK), jnp.int32),
            pltpu.VMEM((SEC, CHUNK), jnp.int32),
            pltpu.VMEM((CNT_R, CHUNK), jnp.float32),
            *([pltpu.VMEM((CHUNK, D), jnp.float32)] * NBUF),
            pltpu.VMEM_SHARED((ACC_ROWS, D), jnp.float32),
            *([pltpu.SemaphoreType.DMA] * (2 * NBUF)),
        ],
        compiler_params=pltpu.CompilerParams(needs_layout_passes=False),
    )


def _seg_pass(tp, tn, gp, sp, gn, sn, zeros):
    return _make_seg_pass()(tp, tn, gp, sp, gn, sn, zeros)


def _pad_idx(idx):
    full = jnp.concatenate(
        [idx, jnp.full((EPPAD - EP,), PAD_IDX, dtype=jnp.int32)])
    return full.reshape(NCHUNK_L, CHUNK)


def kernel(x_var, x_clause, ei_pos, ei_neg, W0vp, b0vp, W0vn, b0vn, W0cp,
           b0cp, W0cn, b0cn, W1vp, b1vp, W1vn, b1vn, W1cp, b1cp, W1cn, b1cn):
    srcP = _pad_idx(ei_pos[0])
    dstP = _pad_idx(ei_pos[1])
    srcN = _pad_idx(ei_neg[0])
    dstN = _pad_idx(ei_neg[1])
    zeros = jnp.zeros((CHUNK, D), jnp.float32)

    x_pad = jnp.pad(x_var, ((0, NPAD - NV), (0, 0)))
    xc_pad = jnp.pad(x_clause, ((0, NPAD - NCL), (0, 8 - x_clause.shape[1])))

    t_p, t_n = _tables_from_x(
        x_pad, W1vp, b1vp.reshape(1, D), W1vn, b1vn.reshape(1, D))
    acc_c, deg_c = _seg_pass(t_p, t_n, srcP, dstP, srcN, dstN, zeros)

    t2_p, t2_n = _tables_from_acc(
        acc_c, deg_c.reshape(2 * NSUB, GRID, BR // CHUNK, CHUNK), xc_pad,
        W1cp[:D], W1cp[D:D + 1], b1cp.reshape(1, D),
        W1cn[:D], W1cn[D:D + 1], b1cn.reshape(1, D))
    acc_v, deg_v = _seg_pass(t2_p, t2_n, dstP, srcP, dstN, srcN, zeros)

    return _final(
        acc_v, deg_v.reshape(2 * NSUB, GRID, BR // CHUNK, CHUNK))[:NV]
